# baseline (speedup 1.0000x reference)
import functools

import jax
import jax.numpy as jnp
from jax import lax
from jax.experimental import pallas as pl
from jax.experimental.pallas import tpu as pltpu
from jax.experimental.pallas import tpu_sc as plsc

K = 8192
D = 256
B = 16384
COMMIT = 0.25

TB = 1024
NTB = K // TB
NT = NTB * NTB


def _argmin_body(wall_ref, a_ref, wsum_ref, hn2r_ref, rmax_ref, ridx_ref,
                 hl_ref):
    t = pl.program_id(0)

    @pl.when(t == 0)
    def _():
        wb0 = wall_ref[...]
        sq = wb0 * wb0
        ones = jnp.ones((1, D), jnp.bfloat16)
        n2r = lax.dot_general(ones, sq, (((1,), (1,)), ((), ())),
                              preferred_element_type=jnp.float32)
        wsum_ref[0, 0] = jnp.sum(n2r)
        hn2r_ref[...] = 0.5 * n2r
        r_io = lax.broadcasted_iota(jnp.int32, (TB, 2), 0)
        c_io = lax.broadcasted_iota(jnp.int32, (TB, 2), 1)
        hl_ref[...] = jnp.where(c_io == 0, r_io // 64, r_io % 64
                                ).astype(jnp.bfloat16)

    @pl.when(t < NT)
    def _():
        bi = (t // NTB) * TB
        bj = (t % NTB) * TB
        xi = wall_ref[pl.ds(bi, TB), :]
        xj = wall_ref[pl.ds(bj, TB), :]
        s = lax.dot_general(xi, xj, (((1,), (1,)), ((), ())),
                            preferred_element_type=jnp.float32)
        sc1 = s - hn2r_ref[0:1, pl.ds(bj, TB)]
        m1 = jnp.max(sc1, axis=1, keepdims=True)
        match1 = (sc1 >= m1).astype(jnp.bfloat16)
        h1 = lax.dot_general(match1, hl_ref[...], (((1,), (0,)), ((), ())),
                             preferred_element_type=jnp.float32)
        i1 = (h1[:, 0:1] * 64.0 + h1[:, 1:2]).astype(jnp.int32) + bj
        old = rmax_ref[pl.ds(bi, TB), :]
        oldi = ridx_ref[pl.ds(bi, TB), :]
        first = bj == 0
        b1 = jnp.logical_or(m1 > old, first)
        rmax_ref[pl.ds(bi, TB), :] = jnp.where(b1, m1, old)
        ridx_ref[pl.ds(bi, TB), :] = jnp.where(b1, i1, oldi)

    @pl.when(t == NT)
    def _():
        a_ref[...] = ridx_ref[...]


def _codebook_argmin(wb):
    return pl.pallas_call(
        _argmin_body,
        grid=(NT + 1,),
        in_specs=[
            pl.BlockSpec((K, D), lambda i: (0, 0)),
        ],
        out_specs=[
            pl.BlockSpec((K, 1), lambda i: (0, 0)),
            pl.BlockSpec(memory_space=pltpu.SMEM),
        ],
        out_shape=[
            jax.ShapeDtypeStruct((K, 1), jnp.int32),
            jax.ShapeDtypeStruct((1, 1), jnp.float32),
        ],
        scratch_shapes=[
            pltpu.VMEM((1, K), jnp.float32),
            pltpu.VMEM((K, 1), jnp.float32),
            pltpu.VMEM((K, 1), jnp.int32),
            pltpu.VMEM((TB, 2), jnp.bfloat16),
        ],
    )(wb)


_NW = 32
_BPW = B // _NW
_NCH = 4
_CH = _BPW // _NCH
_KPW = K // _NW
_KCH = _KPW // _CH


def _wq_body(a_hbm, w_hbm, wq_hbm, idx_v, rows_v, rows2_v, gs0, gs1, ws0, ws1):
    wid = lax.axis_index("s") * 2 + lax.axis_index("c")
    base = wid * _KPW
    bufs, gsem, wsem = (rows_v, rows2_v), (gs0, gs1), (ws0, ws1)
    for j in range(_KCH):
        pltpu.sync_copy(a_hbm.at[pl.ds(base + j * _CH, _CH)], idx_v.at[j])
    g = [pltpu.async_copy(w_hbm.at[idx_v.at[j]], bufs[j], gsem[j])
         for j in range(_KCH)]
    w = []
    for j in range(_KCH):
        g[j].wait()
        w.append(pltpu.async_copy(
            bufs[j], wq_hbm.at[pl.ds(base + j * _CH, _CH)], wsem[j]))
    for c in w:
        c.wait()


_DCH = 64
_DN = _BPW // _DCH


_NBUF = 3


def _fused_body(x_hbm, w_hbm, wq_hbm, q_hbm, d_hbm, p_hbm,
                xidx_v, e0, e1, e2, q0, q1, q2, acc_v,
                ge0, ge1, ge2, gq0, gq1, gq2, we0, we1, we2, wq0, wq1, wq2):
    wid = lax.axis_index("s") * 2 + lax.axis_index("c")
    base = wid * _BPW
    ebufs, qbufs = (e0, e1, e2), (q0, q1, q2)
    gesem, gqsem = (ge0, ge1, ge2), (gq0, gq1, gq2)
    wesem, wqsem = (we0, we1, we2), (wq0, wq1, wq2)
    for j in range(_DN):
        pltpu.sync_copy(x_hbm.at[pl.ds(base + j * _DCH, _DCH)], xidx_v.at[j])
    ge = [pltpu.async_copy(w_hbm.at[xidx_v.at[j]], ebufs[j], gesem[j])
          for j in range(_NBUF)]
    gq = [pltpu.async_copy(wq_hbm.at[xidx_v.at[j]], qbufs[j], gqsem[j])
          for j in range(_NBUF)]
    acc = jnp.zeros((16,), jnp.float32)
    we, wq = [], []
    for j in range(_DN):
        b = j % _NBUF
        ge[j].wait()
        gq[j].wait()
        eb, qb = ebufs[b], qbufs[b]

        def row_body(r, a2, eb=eb, qb=qb):
            for c in range(D // 16):
                ev = eb[r, pl.ds(c * 16, 16)]
                qv = qb[r, pl.ds(c * 16, 16)]
                dv = qv - ev
                eb[r, pl.ds(c * 16, 16)] = dv
                a2 = a2 + dv * dv
            return a2

        acc = lax.fori_loop(0, _DCH, row_body, acc)
        wq.append(pltpu.async_copy(
            qb, q_hbm.at[pl.ds(base + j * _DCH, _DCH)], wqsem[b]))
        we.append(pltpu.async_copy(
            eb, d_hbm.at[pl.ds(base + j * _DCH, _DCH)], wesem[b]))
        if j + _NBUF < _DN:
            we[j].wait()
            wq[j].wait()
            ge.append(pltpu.async_copy(
                w_hbm.at[xidx_v.at[j + _NBUF]], ebufs[b], gesem[b]))
            gq.append(pltpu.async_copy(
                wq_hbm.at[xidx_v.at[j + _NBUF]], qbufs[b], gqsem[b]))
    for j in range(_DN - _NBUF, _DN):
        we[j].wait()
        wq[j].wait()
    acc_v[...] = acc
    pltpu.sync_copy(acc_v, p_hbm.at[wid])


@functools.cache
def _wq_gather():
    return pl.kernel(
        _wq_body,
        out_type=jax.ShapeDtypeStruct((K, D), jnp.float32),
        mesh=plsc.VectorSubcoreMesh(core_axis_name="c", subcore_axis_name="s"),
        scratch_types=[
            pltpu.VMEM((_KCH, _CH), jnp.int32),
            pltpu.VMEM((_CH, D), jnp.float32),
            pltpu.VMEM((_CH, D), jnp.float32),
            pltpu.SemaphoreType.DMA,
            pltpu.SemaphoreType.DMA,
            pltpu.SemaphoreType.DMA,
            pltpu.SemaphoreType.DMA,
        ],
    )


@functools.cache
def _fused_gather():
    return pl.kernel(
        _fused_body,
        out_type=[
            jax.ShapeDtypeStruct((B, D), jnp.float32),
            jax.ShapeDtypeStruct((B, D), jnp.float32),
            jax.ShapeDtypeStruct((_NW, 16), jnp.float32),
        ],
        mesh=plsc.VectorSubcoreMesh(core_axis_name="c", subcore_axis_name="s"),
        scratch_types=(
            [pltpu.VMEM((_DN, _DCH), jnp.int32)]
            + [pltpu.VMEM((_DCH, D), jnp.float32)] * 6
            + [pltpu.VMEM((16,), jnp.float32)]
            + [pltpu.SemaphoreType.DMA] * 12
        ),
    )


DB = 512


def _loss_body(wsum_ref, p_ref, loss_ref):
    loss_ref[0, 0] = jnp.sum(p_ref[...]) / B + COMMIT * wsum_ref[0, 0]


def _loss_combine(wsum, parts):
    return pl.pallas_call(
        _loss_body,
        in_specs=[
            pl.BlockSpec(memory_space=pltpu.SMEM),
            pl.BlockSpec((_NW, 16), lambda: (0, 0)),
        ],
        out_specs=pl.BlockSpec(memory_space=pltpu.SMEM),
        out_shape=jax.ShapeDtypeStruct((1, 1), jnp.float32),
    )(wsum, parts)


def kernel(x, W):
    xi = x.astype(jnp.int32)
    wb = W.astype(jnp.bfloat16)
    a, wsum = _codebook_argmin(wb)
    wq = _wq_gather()(a.reshape(K), W)
    q, diff, parts = _fused_gather()(xi, W, wq)
    loss = _loss_combine(wsum, parts)
    return (loss[0, 0], q, diff)


# speedup vs baseline: 1.4419x; 1.4419x over previous
import functools

import jax
import jax.numpy as jnp
from jax import lax
from jax.experimental import pallas as pl
from jax.experimental.pallas import tpu as pltpu
from jax.experimental.pallas import tpu_sc as plsc

K = 8192
D = 256
B = 16384
COMMIT = 0.25

TB = 1024
NTB = K // TB
_PAIRS = [(i, j) for i in range(NTB) for j in range(i, NTB)]
NT = len(_PAIRS)


def _argmin_body(ii_ref, jj_ref, wall_ref, a_ref, wsum_ref,
                 hn2r_ref, hn2c_ref, rmax_ref, ridx_ref, cmax_ref, cidx_ref,
                 hl_ref):
    t = pl.program_id(0)

    @pl.when(t == 0)
    def _():
        wb0 = wall_ref[...]
        sq = wb0 * wb0
        ones = jnp.ones((1, D), jnp.bfloat16)
        n2r = lax.dot_general(ones, sq, (((1,), (1,)), ((), ())),
                              preferred_element_type=jnp.float32)
        wsum_ref[0, 0] = jnp.sum(n2r)
        hn2r_ref[...] = 0.5 * n2r
        hn2c_ref[...] = jnp.reshape(0.5 * n2r, (K, 1))
        rmax_ref[...] = jnp.full((K, 1), -3e38, jnp.float32)
        cmax_ref[...] = jnp.full((1, K), -3e38, jnp.float32)
        r_io = lax.broadcasted_iota(jnp.int32, (TB, 2), 0)
        c_io = lax.broadcasted_iota(jnp.int32, (TB, 2), 1)
        hl_ref[...] = jnp.where(c_io == 0, r_io // 64, r_io % 64
                                ).astype(jnp.bfloat16)

    @pl.when(t < NT)
    def _():
        bi = ii_ref[t] * TB
        bj = jj_ref[t] * TB
        xi = wall_ref[pl.ds(bi, TB), :]
        xj = wall_ref[pl.ds(bj, TB), :]
        s = lax.dot_general(xi, xj, (((1,), (1,)), ((), ())),
                            preferred_element_type=jnp.float32)
        hl = hl_ref[...]
        sc1 = s - hn2r_ref[0:1, pl.ds(bj, TB)]
        m1 = jnp.max(sc1, axis=1, keepdims=True)
        match1 = (sc1 >= m1).astype(jnp.bfloat16)
        h1 = lax.dot_general(match1, hl, (((1,), (0,)), ((), ())),
                             preferred_element_type=jnp.float32)
        i1 = (h1[:, 0:1] * 64.0 + h1[:, 1:2]).astype(jnp.int32) + bj
        old = rmax_ref[pl.ds(bi, TB), :]
        oldi = ridx_ref[pl.ds(bi, TB), :]
        b1 = m1 > old
        rmax_ref[pl.ds(bi, TB), :] = jnp.where(b1, m1, old)
        ridx_ref[pl.ds(bi, TB), :] = jnp.where(b1, i1, oldi)
        sc2 = s - hn2c_ref[pl.ds(bi, TB), :]
        m0 = jnp.max(sc2, axis=0, keepdims=True)
        match0 = (sc2 >= m0).astype(jnp.bfloat16)
        h0 = lax.dot_general(hl, match0, (((0,), (0,)), ((), ())),
                             preferred_element_type=jnp.float32)
        i0 = (h0[0:1, :] * 64.0 + h0[1:2, :]).astype(jnp.int32) + bi
        oldc = cmax_ref[0:1, pl.ds(bj, TB)]
        oldci = cidx_ref[0:1, pl.ds(bj, TB)]
        b0 = m0 > oldc
        cmax_ref[0:1, pl.ds(bj, TB)] = jnp.where(b0, m0, oldc)
        cidx_ref[0:1, pl.ds(bj, TB)] = jnp.where(b0, i0, oldci)

    @pl.when(t == NT)
    def _():
        cm = jnp.reshape(cmax_ref[...], (K, 1))
        ci = jnp.reshape(cidx_ref[...], (K, 1))
        a_ref[...] = jnp.where(cm > rmax_ref[...], ci, ridx_ref[...])


def _codebook_argmin(wb):
    ii = jnp.asarray([p[0] for p in _PAIRS] + [0], jnp.int32)
    jj = jnp.asarray([p[1] for p in _PAIRS] + [0], jnp.int32)
    return pl.pallas_call(
        _argmin_body,
        grid=(NT + 1,),
        in_specs=[
            pl.BlockSpec(memory_space=pltpu.SMEM),
            pl.BlockSpec(memory_space=pltpu.SMEM),
            pl.BlockSpec((K, D), lambda i: (0, 0)),
        ],
        out_specs=[
            pl.BlockSpec((K, 1), lambda i: (0, 0)),
            pl.BlockSpec(memory_space=pltpu.SMEM),
        ],
        out_shape=[
            jax.ShapeDtypeStruct((K, 1), jnp.int32),
            jax.ShapeDtypeStruct((1, 1), jnp.float32),
        ],
        scratch_shapes=[
            pltpu.VMEM((1, K), jnp.float32),
            pltpu.VMEM((K, 1), jnp.float32),
            pltpu.VMEM((K, 1), jnp.float32),
            pltpu.VMEM((K, 1), jnp.int32),
            pltpu.VMEM((1, K), jnp.float32),
            pltpu.VMEM((1, K), jnp.int32),
            pltpu.VMEM((TB, 2), jnp.bfloat16),
        ],
    )(ii, jj, wb)


_NW = 32
_BPW = B // _NW
_NCH = 4
_CH = _BPW // _NCH
_KPW = K // _NW
_KCH = _KPW // _CH


def _wq_body(a_hbm, w_hbm, wq_hbm, idx_v, rows_v, rows2_v, gs0, gs1, ws0, ws1):
    wid = lax.axis_index("s") * 2 + lax.axis_index("c")
    base = wid * _KPW
    bufs, gsem, wsem = (rows_v, rows2_v), (gs0, gs1), (ws0, ws1)
    for j in range(_KCH):
        pltpu.sync_copy(a_hbm.at[pl.ds(base + j * _CH, _CH)], idx_v.at[j])
    g = [pltpu.async_copy(w_hbm.at[idx_v.at[j]], bufs[j], gsem[j])
         for j in range(_KCH)]
    w = []
    for j in range(_KCH):
        g[j].wait()
        w.append(pltpu.async_copy(
            bufs[j], wq_hbm.at[pl.ds(base + j * _CH, _CH)], wsem[j]))
    for c in w:
        c.wait()


_DCH = 64
_DN = _BPW // _DCH


_NBUF = 3


def _fused_body(x_hbm, w_hbm, wq_hbm, q_hbm, d_hbm, p_hbm,
                xidx_v, e0, e1, e2, q0, q1, q2, acc_v,
                ge0, ge1, ge2, gq0, gq1, gq2, we0, we1, we2, wq0, wq1, wq2):
    wid = lax.axis_index("s") * 2 + lax.axis_index("c")
    base = wid * _BPW
    ebufs, qbufs = (e0, e1, e2), (q0, q1, q2)
    gesem, gqsem = (ge0, ge1, ge2), (gq0, gq1, gq2)
    wesem, wqsem = (we0, we1, we2), (wq0, wq1, wq2)
    for j in range(_DN):
        pltpu.sync_copy(x_hbm.at[pl.ds(base + j * _DCH, _DCH)], xidx_v.at[j])
    ge = [pltpu.async_copy(w_hbm.at[xidx_v.at[j]], ebufs[j], gesem[j])
          for j in range(_NBUF)]
    gq = [pltpu.async_copy(wq_hbm.at[xidx_v.at[j]], qbufs[j], gqsem[j])
          for j in range(_NBUF)]
    acc = jnp.zeros((16,), jnp.float32)
    we, wq = [], []
    for j in range(_DN):
        b = j % _NBUF
        ge[j].wait()
        gq[j].wait()
        eb, qb = ebufs[b], qbufs[b]

        def row_body(r, a2, eb=eb, qb=qb):
            for c in range(D // 16):
                ev = eb[r, pl.ds(c * 16, 16)]
                qv = qb[r, pl.ds(c * 16, 16)]
                dv = qv - ev
                eb[r, pl.ds(c * 16, 16)] = dv
                a2 = a2 + dv * dv
            return a2

        acc = lax.fori_loop(0, _DCH, row_body, acc)
        wq.append(pltpu.async_copy(
            qb, q_hbm.at[pl.ds(base + j * _DCH, _DCH)], wqsem[b]))
        we.append(pltpu.async_copy(
            eb, d_hbm.at[pl.ds(base + j * _DCH, _DCH)], wesem[b]))
        if j + _NBUF < _DN:
            we[j].wait()
            wq[j].wait()
            ge.append(pltpu.async_copy(
                w_hbm.at[xidx_v.at[j + _NBUF]], ebufs[b], gesem[b]))
            gq.append(pltpu.async_copy(
                wq_hbm.at[xidx_v.at[j + _NBUF]], qbufs[b], gqsem[b]))
    for j in range(_DN - _NBUF, _DN):
        we[j].wait()
        wq[j].wait()
    acc_v[...] = acc
    pltpu.sync_copy(acc_v, p_hbm.at[wid])


@functools.cache
def _wq_gather():
    return pl.kernel(
        _wq_body,
        out_type=jax.ShapeDtypeStruct((K, D), jnp.float32),
        mesh=plsc.VectorSubcoreMesh(core_axis_name="c", subcore_axis_name="s"),
        scratch_types=[
            pltpu.VMEM((_KCH, _CH), jnp.int32),
            pltpu.VMEM((_CH, D), jnp.float32),
            pltpu.VMEM((_CH, D), jnp.float32),
            pltpu.SemaphoreType.DMA,
            pltpu.SemaphoreType.DMA,
            pltpu.SemaphoreType.DMA,
            pltpu.SemaphoreType.DMA,
        ],
    )


@functools.cache
def _fused_gather():
    return pl.kernel(
        _fused_body,
        out_type=[
            jax.ShapeDtypeStruct((B, D), jnp.float32),
            jax.ShapeDtypeStruct((B, D), jnp.float32),
            jax.ShapeDtypeStruct((_NW, 16), jnp.float32),
        ],
        mesh=plsc.VectorSubcoreMesh(core_axis_name="c", subcore_axis_name="s"),
        scratch_types=(
            [pltpu.VMEM((_DN, _DCH), jnp.int32)]
            + [pltpu.VMEM((_DCH, D), jnp.float32)] * 6
            + [pltpu.VMEM((16,), jnp.float32)]
            + [pltpu.SemaphoreType.DMA] * 12
        ),
    )


DB = 512


def _loss_body(wsum_ref, p_ref, loss_ref):
    loss_ref[0, 0] = jnp.sum(p_ref[...]) / B + COMMIT * wsum_ref[0, 0]


def _loss_combine(wsum, parts):
    return pl.pallas_call(
        _loss_body,
        in_specs=[
            pl.BlockSpec(memory_space=pltpu.SMEM),
            pl.BlockSpec((_NW, 16), lambda: (0, 0)),
        ],
        out_specs=pl.BlockSpec(memory_space=pltpu.SMEM),
        out_shape=jax.ShapeDtypeStruct((1, 1), jnp.float32),
    )(wsum, parts)


def kernel(x, W):
    xi = x.astype(jnp.int32)
    wb = W.astype(jnp.bfloat16)
    a, wsum = _codebook_argmin(wb)
    wq = _wq_gather()(a.reshape(K), W)
    q, diff, parts = _fused_gather()(xi, W, wq)
    loss = _loss_combine(wsum, parts)
    return (loss[0, 0], q, diff)


# speedup vs baseline: 1.4725x; 1.0212x over previous
import functools

import jax
import jax.numpy as jnp
from jax import lax
from jax.experimental import pallas as pl
from jax.experimental.pallas import tpu as pltpu
from jax.experimental.pallas import tpu_sc as plsc

K = 8192
D = 256
B = 16384
COMMIT = 0.25

TB = 1024
NTB = K // TB
_PAIRS = [(i, j) for i in range(NTB) for j in range(i, NTB)]
NT = len(_PAIRS)


def _argmin_body(ii_ref, jj_ref, wall_ref, a_ref, wsum_ref,
                 hn2r_ref, hn2c_ref, rmax_ref, ridx_ref, cmax_ref, cidx_ref,
                 hl_ref):
    t = pl.program_id(0)

    @pl.when(t == 0)
    def _():
        wb0 = wall_ref[...]
        sq = wb0 * wb0
        ones = jnp.ones((1, D), jnp.bfloat16)
        n2r = lax.dot_general(ones, sq, (((1,), (1,)), ((), ())),
                              preferred_element_type=jnp.float32)
        wsum_ref[0, 0] = jnp.sum(n2r)
        hn2r_ref[...] = 0.5 * n2r
        hn2c_ref[...] = jnp.reshape(0.5 * n2r, (K, 1))
        rmax_ref[...] = jnp.full((K, 1), -3e38, jnp.float32)
        cmax_ref[...] = jnp.full((1, K), -3e38, jnp.float32)
        r_io = lax.broadcasted_iota(jnp.int32, (TB, 2), 0)
        c_io = lax.broadcasted_iota(jnp.int32, (TB, 2), 1)
        hl_ref[...] = jnp.where(c_io == 0, r_io // 64, r_io % 64
                                ).astype(jnp.bfloat16)

    @pl.when(t < NT)
    def _():
        bi = ii_ref[t] * TB
        bj = jj_ref[t] * TB
        xi = wall_ref[pl.ds(bi, TB), :]
        xj = wall_ref[pl.ds(bj, TB), :]
        s = lax.dot_general(xi, xj, (((1,), (1,)), ((), ())),
                            preferred_element_type=jnp.float32)
        hl = hl_ref[...]
        sc1 = s - hn2r_ref[0:1, pl.ds(bj, TB)]
        m1 = jnp.max(sc1, axis=1, keepdims=True)
        match1 = (sc1 >= m1).astype(jnp.bfloat16)
        h1 = lax.dot_general(match1, hl, (((1,), (0,)), ((), ())),
                             preferred_element_type=jnp.float32)
        i1 = (h1[:, 0:1] * 64.0 + h1[:, 1:2]).astype(jnp.int32) + bj
        old = rmax_ref[pl.ds(bi, TB), :]
        oldi = ridx_ref[pl.ds(bi, TB), :]
        b1 = m1 > old
        rmax_ref[pl.ds(bi, TB), :] = jnp.where(b1, m1, old)
        ridx_ref[pl.ds(bi, TB), :] = jnp.where(b1, i1, oldi)
        sc2 = s - hn2c_ref[pl.ds(bi, TB), :]
        m0 = jnp.max(sc2, axis=0, keepdims=True)
        match0 = (sc2 >= m0).astype(jnp.bfloat16)
        h0 = lax.dot_general(hl, match0, (((0,), (0,)), ((), ())),
                             preferred_element_type=jnp.float32)
        i0 = (h0[0:1, :] * 64.0 + h0[1:2, :]).astype(jnp.int32) + bi
        oldc = cmax_ref[0:1, pl.ds(bj, TB)]
        oldci = cidx_ref[0:1, pl.ds(bj, TB)]
        b0 = m0 > oldc
        cmax_ref[0:1, pl.ds(bj, TB)] = jnp.where(b0, m0, oldc)
        cidx_ref[0:1, pl.ds(bj, TB)] = jnp.where(b0, i0, oldci)

    @pl.when(t == NT)
    def _():
        cm = jnp.reshape(cmax_ref[...], (K, 1))
        ci = jnp.reshape(cidx_ref[...], (K, 1))
        a_ref[...] = jnp.where(cm > rmax_ref[...], ci, ridx_ref[...])


def _codebook_argmin(wb):
    ii = jnp.asarray([p[0] for p in _PAIRS] + [0], jnp.int32)
    jj = jnp.asarray([p[1] for p in _PAIRS] + [0], jnp.int32)
    return pl.pallas_call(
        _argmin_body,
        grid=(NT + 1,),
        in_specs=[
            pl.BlockSpec(memory_space=pltpu.SMEM),
            pl.BlockSpec(memory_space=pltpu.SMEM),
            pl.BlockSpec((K, D), lambda i: (0, 0)),
        ],
        out_specs=[
            pl.BlockSpec((K, 1), lambda i: (0, 0)),
            pl.BlockSpec(memory_space=pltpu.SMEM),
        ],
        out_shape=[
            jax.ShapeDtypeStruct((K, 1), jnp.int32),
            jax.ShapeDtypeStruct((1, 1), jnp.float32),
        ],
        scratch_shapes=[
            pltpu.VMEM((1, K), jnp.float32),
            pltpu.VMEM((K, 1), jnp.float32),
            pltpu.VMEM((K, 1), jnp.float32),
            pltpu.VMEM((K, 1), jnp.int32),
            pltpu.VMEM((1, K), jnp.float32),
            pltpu.VMEM((1, K), jnp.int32),
            pltpu.VMEM((TB, 2), jnp.bfloat16),
        ],
    )(ii, jj, wb)


_NW = 32
_BPW = B // _NW
_NCH = 4
_CH = _BPW // _NCH
_KPW = K // _NW
_KCH = _KPW // _CH


def _wq_body(a_hbm, w_hbm, wq_hbm, idx_v, rows_v, rows2_v, gs0, gs1, ws0, ws1):
    wid = lax.axis_index("s") * 2 + lax.axis_index("c")
    base = wid * _KPW
    bufs, gsem, wsem = (rows_v, rows2_v), (gs0, gs1), (ws0, ws1)
    pltpu.sync_copy(a_hbm.at[pl.ds(base, _KPW)], idx_v)
    g = [pltpu.async_copy(w_hbm.at[idx_v.at[pl.ds(j * _CH, _CH)]],
                          bufs[j], gsem[j])
         for j in range(_KCH)]
    w = []
    for j in range(_KCH):
        g[j].wait()
        w.append(pltpu.async_copy(
            bufs[j], wq_hbm.at[pl.ds(base + j * _CH, _CH)], wsem[j]))
    for c in w:
        c.wait()


_DCH = 64
_DN = _BPW // _DCH


_NBUF = 3


def _fused_body(x_hbm, w_hbm, wq_hbm, q_hbm, d_hbm, p_hbm,
                xidx_v, e0, e1, e2, q0, q1, q2, acc_v,
                ge0, ge1, ge2, gq0, gq1, gq2, we0, we1, we2, wq0, wq1, wq2):
    wid = lax.axis_index("s") * 2 + lax.axis_index("c")
    base = wid * _BPW
    ebufs, qbufs = (e0, e1, e2), (q0, q1, q2)
    gesem, gqsem = (ge0, ge1, ge2), (gq0, gq1, gq2)
    wesem, wqsem = (we0, we1, we2), (wq0, wq1, wq2)
    pltpu.sync_copy(x_hbm.at[pl.ds(base, _BPW)], xidx_v)
    idx = [xidx_v.at[pl.ds(j * _DCH, _DCH)] for j in range(_DN)]
    ge = [pltpu.async_copy(w_hbm.at[idx[j]], ebufs[j], gesem[j])
          for j in range(_NBUF)]
    gq = [pltpu.async_copy(wq_hbm.at[idx[j]], qbufs[j], gqsem[j])
          for j in range(_NBUF)]
    acc = jnp.zeros((16,), jnp.float32)
    we, wq = [], []
    for j in range(_DN):
        b = j % _NBUF
        ge[j].wait()
        gq[j].wait()
        eb, qb = ebufs[b], qbufs[b]

        def row_body(r, a2, eb=eb, qb=qb):
            for c in range(D // 16):
                ev = eb[r, pl.ds(c * 16, 16)]
                qv = qb[r, pl.ds(c * 16, 16)]
                dv = qv - ev
                eb[r, pl.ds(c * 16, 16)] = dv
                a2 = a2 + dv * dv
            return a2

        acc = lax.fori_loop(0, _DCH, row_body, acc)
        wq.append(pltpu.async_copy(
            qb, q_hbm.at[pl.ds(base + j * _DCH, _DCH)], wqsem[b]))
        we.append(pltpu.async_copy(
            eb, d_hbm.at[pl.ds(base + j * _DCH, _DCH)], wesem[b]))
        if j + _NBUF < _DN:
            we[j].wait()
            wq[j].wait()
            ge.append(pltpu.async_copy(
                w_hbm.at[idx[j + _NBUF]], ebufs[b], gesem[b]))
            gq.append(pltpu.async_copy(
                wq_hbm.at[idx[j + _NBUF]], qbufs[b], gqsem[b]))
    for j in range(_DN - _NBUF, _DN):
        we[j].wait()
        wq[j].wait()
    acc_v[...] = acc
    pltpu.sync_copy(acc_v, p_hbm.at[wid])


@functools.cache
def _wq_gather():
    return pl.kernel(
        _wq_body,
        out_type=jax.ShapeDtypeStruct((K, D), jnp.float32),
        mesh=plsc.VectorSubcoreMesh(core_axis_name="c", subcore_axis_name="s"),
        scratch_types=[
            pltpu.VMEM((_KPW,), jnp.int32),
            pltpu.VMEM((_CH, D), jnp.float32),
            pltpu.VMEM((_CH, D), jnp.float32),
            pltpu.SemaphoreType.DMA,
            pltpu.SemaphoreType.DMA,
            pltpu.SemaphoreType.DMA,
            pltpu.SemaphoreType.DMA,
        ],
    )


@functools.cache
def _fused_gather():
    return pl.kernel(
        _fused_body,
        out_type=[
            jax.ShapeDtypeStruct((B, D), jnp.float32),
            jax.ShapeDtypeStruct((B, D), jnp.float32),
            jax.ShapeDtypeStruct((_NW, 16), jnp.float32),
        ],
        mesh=plsc.VectorSubcoreMesh(core_axis_name="c", subcore_axis_name="s"),
        scratch_types=(
            [pltpu.VMEM((_BPW,), jnp.int32)]
            + [pltpu.VMEM((_DCH, D), jnp.float32)] * 6
            + [pltpu.VMEM((16,), jnp.float32)]
            + [pltpu.SemaphoreType.DMA] * 12
        ),
    )


DB = 512


def _loss_body(wsum_ref, p_ref, loss_ref):
    loss_ref[0, 0] = jnp.sum(p_ref[...]) / B + COMMIT * wsum_ref[0, 0]


def _loss_combine(wsum, parts):
    return pl.pallas_call(
        _loss_body,
        in_specs=[
            pl.BlockSpec(memory_space=pltpu.SMEM),
            pl.BlockSpec((_NW, 16), lambda: (0, 0)),
        ],
        out_specs=pl.BlockSpec(memory_space=pltpu.SMEM),
        out_shape=jax.ShapeDtypeStruct((1, 1), jnp.float32),
    )(wsum, parts)


def kernel(x, W):
    xi = x.astype(jnp.int32)
    wb = W.astype(jnp.bfloat16)
    a, wsum = _codebook_argmin(wb)
    wq = _wq_gather()(a.reshape(K), W)
    q, diff, parts = _fused_gather()(xi, W, wq)
    loss = _loss_combine(wsum, parts)
    return (loss[0, 0], q, diff)


# speedup vs baseline: 1.4734x; 1.0006x over previous
import functools

import jax
import jax.numpy as jnp
from jax import lax
from jax.experimental import pallas as pl
from jax.experimental.pallas import tpu as pltpu
from jax.experimental.pallas import tpu_sc as plsc

K = 8192
D = 256
B = 16384
COMMIT = 0.25

TB = 1024
NTB = K // TB
_PAIRS = [(i, j) for i in range(NTB) for j in range(i, NTB)]
NT = len(_PAIRS)


def _argmin_body(ii_ref, jj_ref, wall_ref, a_ref, wsum_ref,
                 hn2r_ref, hn2c_ref, rmax_ref, ridx_ref, cmax_ref, cidx_ref,
                 hl_ref):
    t = pl.program_id(0)

    @pl.when(t == 0)
    def _():
        wb0 = wall_ref[...]
        sq = wb0 * wb0
        ones = jnp.ones((1, D), jnp.bfloat16)
        n2r = lax.dot_general(ones, sq, (((1,), (1,)), ((), ())),
                              preferred_element_type=jnp.float32)
        wsum_ref[0, 0] = jnp.sum(n2r)
        hn2r_ref[...] = 0.5 * n2r
        hn2c_ref[...] = jnp.reshape(0.5 * n2r, (K, 1))
        rmax_ref[...] = jnp.full((K, 1), -3e38, jnp.float32)
        cmax_ref[...] = jnp.full((1, K), -3e38, jnp.float32)
        r_io = lax.broadcasted_iota(jnp.int32, (TB, 2), 0)
        c_io = lax.broadcasted_iota(jnp.int32, (TB, 2), 1)
        hl_ref[...] = jnp.where(c_io == 0, r_io // 64, r_io % 64
                                ).astype(jnp.bfloat16)

    @pl.when(t < NT)
    def _():
        bi = ii_ref[t] * TB
        bj = jj_ref[t] * TB
        xi = wall_ref[pl.ds(bi, TB), :]
        xj = wall_ref[pl.ds(bj, TB), :]
        s = lax.dot_general(xi, xj, (((1,), (1,)), ((), ())),
                            preferred_element_type=jnp.float32)
        hl = hl_ref[...]
        sc1 = s - hn2r_ref[0:1, pl.ds(bj, TB)]
        m1 = jnp.max(sc1, axis=1, keepdims=True)
        match1 = (sc1 >= m1).astype(jnp.bfloat16)
        h1 = lax.dot_general(match1, hl, (((1,), (0,)), ((), ())),
                             preferred_element_type=jnp.float32)
        i1 = (h1[:, 0:1] * 64.0 + h1[:, 1:2]).astype(jnp.int32) + bj
        old = rmax_ref[pl.ds(bi, TB), :]
        oldi = ridx_ref[pl.ds(bi, TB), :]
        b1 = m1 > old
        rmax_ref[pl.ds(bi, TB), :] = jnp.where(b1, m1, old)
        ridx_ref[pl.ds(bi, TB), :] = jnp.where(b1, i1, oldi)
        sc2 = s - hn2c_ref[pl.ds(bi, TB), :]
        m0 = jnp.max(sc2, axis=0, keepdims=True)
        match0 = (sc2 >= m0).astype(jnp.bfloat16)
        h0 = lax.dot_general(hl, match0, (((0,), (0,)), ((), ())),
                             preferred_element_type=jnp.float32)
        i0 = (h0[0:1, :] * 64.0 + h0[1:2, :]).astype(jnp.int32) + bi
        oldc = cmax_ref[0:1, pl.ds(bj, TB)]
        oldci = cidx_ref[0:1, pl.ds(bj, TB)]
        b0 = m0 > oldc
        cmax_ref[0:1, pl.ds(bj, TB)] = jnp.where(b0, m0, oldc)
        cidx_ref[0:1, pl.ds(bj, TB)] = jnp.where(b0, i0, oldci)

    @pl.when(t == NT)
    def _():
        cm = jnp.reshape(cmax_ref[...], (K, 1))
        ci = jnp.reshape(cidx_ref[...], (K, 1))
        a_ref[...] = jnp.where(cm > rmax_ref[...], ci, ridx_ref[...])


def _codebook_argmin(wb):
    ii = jnp.asarray([p[0] for p in _PAIRS] + [0], jnp.int32)
    jj = jnp.asarray([p[1] for p in _PAIRS] + [0], jnp.int32)
    return pl.pallas_call(
        _argmin_body,
        grid=(NT + 1,),
        in_specs=[
            pl.BlockSpec(memory_space=pltpu.SMEM),
            pl.BlockSpec(memory_space=pltpu.SMEM),
            pl.BlockSpec((K, D), lambda i: (0, 0)),
        ],
        out_specs=[
            pl.BlockSpec((K, 1), lambda i: (0, 0)),
            pl.BlockSpec(memory_space=pltpu.SMEM),
        ],
        out_shape=[
            jax.ShapeDtypeStruct((K, 1), jnp.int32),
            jax.ShapeDtypeStruct((1, 1), jnp.float32),
        ],
        scratch_shapes=[
            pltpu.VMEM((1, K), jnp.float32),
            pltpu.VMEM((K, 1), jnp.float32),
            pltpu.VMEM((K, 1), jnp.float32),
            pltpu.VMEM((K, 1), jnp.int32),
            pltpu.VMEM((1, K), jnp.float32),
            pltpu.VMEM((1, K), jnp.int32),
            pltpu.VMEM((TB, 2), jnp.bfloat16),
        ],
    )(ii, jj, wb)


_NW = 32
_BPW = B // _NW
_CH = 128
_KPW = K // _NW
_KCH = _KPW // _CH


def _wq_body(a_hbm, w_hbm, wq_hbm, idx_v, rows_v, rows2_v, gs0, gs1, ws0, ws1):
    wid = lax.axis_index("s") * 2 + lax.axis_index("c")
    base = wid * _KPW
    bufs, gsem, wsem = (rows_v, rows2_v), (gs0, gs1), (ws0, ws1)
    pltpu.sync_copy(a_hbm.at[pl.ds(base, _KPW)], idx_v)
    g = [pltpu.async_copy(w_hbm.at[idx_v.at[pl.ds(j * _CH, _CH)]],
                          bufs[j], gsem[j])
         for j in range(_KCH)]
    w = []
    for j in range(_KCH):
        g[j].wait()
        w.append(pltpu.async_copy(
            bufs[j], wq_hbm.at[pl.ds(base + j * _CH, _CH)], wsem[j]))
    for c in w:
        c.wait()


_DCH = 64
_DN = _BPW // _DCH


_NBUF = 3


def _fused_body(x_hbm, w_hbm, wq_hbm, q_hbm, d_hbm, p_hbm,
                xidx_v, e0, e1, e2, q0, q1, q2, acc_v,
                ge0, ge1, ge2, gq0, gq1, gq2, we0, we1, we2, wq0, wq1, wq2):
    wid = lax.axis_index("s") * 2 + lax.axis_index("c")
    base = wid * _BPW
    ebufs, qbufs = (e0, e1, e2), (q0, q1, q2)
    gesem, gqsem = (ge0, ge1, ge2), (gq0, gq1, gq2)
    wesem, wqsem = (we0, we1, we2), (wq0, wq1, wq2)
    pltpu.sync_copy(x_hbm.at[pl.ds(base, _BPW)], xidx_v)
    idx = [xidx_v.at[pl.ds(j * _DCH, _DCH)] for j in range(_DN)]
    ge = [pltpu.async_copy(w_hbm.at[idx[j]], ebufs[j], gesem[j])
          for j in range(_NBUF)]
    gq = [pltpu.async_copy(wq_hbm.at[idx[j]], qbufs[j], gqsem[j])
          for j in range(_NBUF)]
    acc = jnp.zeros((16,), jnp.float32)
    we, wq = [], []
    for j in range(_DN):
        b = j % _NBUF
        ge[j].wait()
        gq[j].wait()
        eb, qb = ebufs[b], qbufs[b]

        def row_body(r, a2, eb=eb, qb=qb):
            for c in range(D // 16):
                ev = eb[r, pl.ds(c * 16, 16)]
                qv = qb[r, pl.ds(c * 16, 16)]
                dv = qv - ev
                eb[r, pl.ds(c * 16, 16)] = dv
                a2 = a2 + dv * dv
            return a2

        acc = lax.fori_loop(0, _DCH, row_body, acc)
        wq.append(pltpu.async_copy(
            qb, q_hbm.at[pl.ds(base + j * _DCH, _DCH)], wqsem[b]))
        we.append(pltpu.async_copy(
            eb, d_hbm.at[pl.ds(base + j * _DCH, _DCH)], wesem[b]))
        if j + _NBUF < _DN:
            we[j].wait()
            wq[j].wait()
            ge.append(pltpu.async_copy(
                w_hbm.at[idx[j + _NBUF]], ebufs[b], gesem[b]))
            gq.append(pltpu.async_copy(
                wq_hbm.at[idx[j + _NBUF]], qbufs[b], gqsem[b]))
    for j in range(_DN - _NBUF, _DN):
        we[j].wait()
        wq[j].wait()
    acc_v[...] = acc
    pltpu.sync_copy(acc_v, p_hbm.at[wid])


@functools.cache
def _wq_gather():
    return pl.kernel(
        _wq_body,
        out_type=jax.ShapeDtypeStruct((K, D), jnp.float32),
        mesh=plsc.VectorSubcoreMesh(core_axis_name="c", subcore_axis_name="s"),
        scratch_types=[
            pltpu.VMEM((_KPW,), jnp.int32),
            pltpu.VMEM((_CH, D), jnp.float32),
            pltpu.VMEM((_CH, D), jnp.float32),
            pltpu.SemaphoreType.DMA,
            pltpu.SemaphoreType.DMA,
            pltpu.SemaphoreType.DMA,
            pltpu.SemaphoreType.DMA,
        ],
    )


@functools.cache
def _fused_gather():
    return pl.kernel(
        _fused_body,
        out_type=[
            jax.ShapeDtypeStruct((B, D), jnp.float32),
            jax.ShapeDtypeStruct((B, D), jnp.float32),
            jax.ShapeDtypeStruct((_NW, 16), jnp.float32),
        ],
        mesh=plsc.VectorSubcoreMesh(core_axis_name="c", subcore_axis_name="s"),
        scratch_types=(
            [pltpu.VMEM((_BPW,), jnp.int32)]
            + [pltpu.VMEM((_DCH, D), jnp.float32)] * 6
            + [pltpu.VMEM((16,), jnp.float32)]
            + [pltpu.SemaphoreType.DMA] * 12
        ),
    )


def _loss_body(wsum_ref, p_ref, loss_ref):
    loss_ref[0, 0] = jnp.sum(p_ref[...]) / B + COMMIT * wsum_ref[0, 0]


def _loss_combine(wsum, parts):
    return pl.pallas_call(
        _loss_body,
        in_specs=[
            pl.BlockSpec(memory_space=pltpu.SMEM),
            pl.BlockSpec((_NW, 16), lambda: (0, 0)),
        ],
        out_specs=pl.BlockSpec(memory_space=pltpu.SMEM),
        out_shape=jax.ShapeDtypeStruct((1, 1), jnp.float32),
    )(wsum, parts)


def kernel(x, W):
    xi = x.astype(jnp.int32)
    wb = W.astype(jnp.bfloat16)
    a, wsum = _codebook_argmin(wb)
    wq = _wq_gather()(a.reshape(K), W)
    q, diff, parts = _fused_gather()(xi, W, wq)
    loss = _loss_combine(wsum, parts)
    return (loss[0, 0], q, diff)
